# baseline (speedup 1.0000x reference)
import functools

import jax
import jax.numpy as jnp
from jax import lax
from jax.experimental import pallas as pl
from jax.experimental.pallas import tpu as pltpu
from jax.experimental.pallas import tpu_sc as plsc

INPUT_DIM = 1000000
HALF = INPUT_DIM // 2
D = 32

NC = 2
NS = 16
NW = NC * NS
L = 16

BSZ = 16384
HIST = 50
BW = BSZ // NW
SB = 128
NSB = BW // SB


def _embed_kernel():
    mesh = plsc.VectorSubcoreMesh(core_axis_name="c", subcore_axis_name="s")

    @functools.partial(
        pl.kernel,
        out_type=jax.ShapeDtypeStruct((HIST, D // 8, BSZ // SB, 8, SB),
                                      jnp.float32),
        mesh=mesh,
        compiler_params=pltpu.CompilerParams(use_tc_tiling_on_sc=False,
                                             needs_layout_passes=False),
        scratch_types=[
            pltpu.VMEM((BW * HIST,), jnp.int32),
            pltpu.VMEM((2, 2 * BW, D), jnp.float32),
            pltpu.VMEM((2, D // 8, NSB, 8, SB), jnp.float32),
            pltpu.VMEM((2, NSB, SB), jnp.int32),
            pltpu.VMEM((2, NSB, SB), jnp.int32),
            pltpu.VMEM((2, BW), jnp.int32),
            pltpu.SemaphoreType.DMA,
            pltpu.SemaphoreType.DMA,
            pltpu.SemaphoreType.DMA,
        ],
    )
    def k(idx_hbm, e1_hbm, e2_hbm, out_hbm,
          idx_v, rows_v, obuf_v, idx1b, idx2b, rselb, gsem0, gsem1, wsem):
        wid = lax.axis_index("s") * NC + lax.axis_index("c")
        iota = lax.broadcasted_iota(jnp.int32, (L,), 0)
        iota50 = iota * HIST

        pltpu.sync_copy(idx_hbm.at[pl.ds(wid * BW * HIST, BW * HIST)], idx_v)

        def prep(h, slot):
            for g in range(BW // L):
                ivec = iota50 + (g * L * HIST + h)
                idx16 = plsc.load_gather(idx_v, [ivec])
                flip = idx16 >= HALF
                sb, o = g // (SB // L), (g % (SB // L)) * L
                idx1b[slot, sb, pl.ds(o, L)] = jnp.minimum(idx16, HALF - 1)
                idx2b[slot, sb, pl.ds(o, L)] = jnp.maximum(idx16 - HALF, 0)
                rselb[slot, pl.ds(g * L, L)] = (
                    (iota + g * L) + jnp.where(flip, BW, 0))

        def fire_gathers(slot, sem):
            cps = []
            for sb in range(NSB):
                cps.append(pltpu.async_copy(
                    e1_hbm.at[idx1b.at[slot, sb]],
                    rows_v.at[slot, pl.ds(sb * SB, SB)], sem))
                cps.append(pltpu.async_copy(
                    e2_hbm.at[idx2b.at[slot, sb]],
                    rows_v.at[slot, pl.ds(BW + sb * SB, SB)], sem))
            return cps

        def transpose_select(slot):
            for g in range(BW // L):
                rsel16 = rselb[slot, pl.ds(g * L, L)]
                sb, o = g // (SB // L), (g % (SB // L)) * L
                for c in range(D):
                    col = jnp.full((L,), c, jnp.int32)
                    v = plsc.load_gather(rows_v.at[slot], [rsel16, col])
                    obuf_v[slot, c // 8, sb, c % 8, pl.ds(o, L)] = v

        def fire_writes(h, slot):
            return [
                pltpu.async_copy(obuf_v.at[slot, cb],
                                 out_hbm.at[h, cb, pl.ds(wid * NSB, NSB)],
                                 wsem)
                for cb in range(D // 8)
            ]

        def body(p, carry):
            h0 = p * 2
            h1 = h0 + 1
            prep(h0, 0)
            g0 = fire_gathers(0, gsem0)
            prep(h1, 1)
            g1 = fire_gathers(1, gsem1)
            for cp in g0:
                cp.wait()
            transpose_select(0)
            w0 = fire_writes(h0, 0)
            for cp in g1:
                cp.wait()
            transpose_select(1)
            w1 = fire_writes(h1, 1)
            for w in w0 + w1:
                w.wait()
            return carry

        lax.fori_loop(0, HIST // 2, body, 0)

    return k


def kernel(inputs, e1, e2):
    bsz, hist = inputs.shape
    idx = inputs.reshape(bsz * hist).astype(jnp.int32)
    out5 = _embed_kernel()(idx, e1, e2)
    x = out5.transpose(2, 4, 0, 1, 3)
    return x.reshape(bsz, hist, D)


# speedup vs baseline: 1.0023x; 1.0023x over previous
import functools

import jax
import jax.numpy as jnp
from jax import lax
from jax.experimental import pallas as pl
from jax.experimental.pallas import tpu as pltpu
from jax.experimental.pallas import tpu_sc as plsc

INPUT_DIM = 1000000
HALF = INPUT_DIM // 2
D = 32

NC = 2
NS = 16
NW = NC * NS
L = 16

BSZ = 16384
HIST = 50
BW = BSZ // NW
SB = 128
NSB = BW // SB


def _embed_kernel():
    mesh = plsc.VectorSubcoreMesh(core_axis_name="c", subcore_axis_name="s")

    @functools.partial(
        pl.kernel,
        out_type=jax.ShapeDtypeStruct((HIST, D // 8, BSZ // SB, 8, SB),
                                      jnp.float32),
        mesh=mesh,
        compiler_params=pltpu.CompilerParams(use_tc_tiling_on_sc=False,
                                             needs_layout_passes=False),
        scratch_types=[
            pltpu.VMEM((BW * HIST,), jnp.int32),
            pltpu.VMEM((2, 2 * BW, D), jnp.float32),
            pltpu.VMEM((2, D // 8, NSB, 8, SB), jnp.float32),
            pltpu.VMEM((2, NSB, SB), jnp.int32),
            pltpu.VMEM((2, NSB, SB), jnp.int32),
            pltpu.VMEM((2, BW), jnp.int32),
            pltpu.SemaphoreType.DMA,
            pltpu.SemaphoreType.DMA,
            pltpu.SemaphoreType.DMA,
        ],
    )
    def k(idx_hbm, e1_hbm, e2_hbm, out_hbm,
          idx_v, rows_v, obuf_v, idx1b, idx2b, rselb, gsem0, gsem1, wsem):
        wid = lax.axis_index("s") * NC + lax.axis_index("c")
        iota = lax.broadcasted_iota(jnp.int32, (L,), 0)
        iota50 = iota * HIST

        pltpu.sync_copy(idx_hbm.at[pl.ds(wid * BW * HIST, BW * HIST)], idx_v)

        def prep(h, slot):
            for g in range(BW // L):
                ivec = iota50 + (g * L * HIST + h)
                idx16 = plsc.load_gather(idx_v, [ivec])
                flip = idx16 >= HALF
                sb, o = g // (SB // L), (g % (SB // L)) * L
                idx1b[slot, sb, pl.ds(o, L)] = jnp.minimum(idx16, HALF - 1)
                idx2b[slot, sb, pl.ds(o, L)] = jnp.maximum(idx16 - HALF, 0)
                rselb[slot, pl.ds(g * L, L)] = (
                    (iota + g * L) + jnp.where(flip, BW, 0))

        def fire_gathers(slot, sem):
            cps = []
            for sb in range(NSB):
                cps.append(pltpu.async_copy(
                    e1_hbm.at[idx1b.at[slot, sb]],
                    rows_v.at[slot, pl.ds(sb * SB, SB)], sem))
                cps.append(pltpu.async_copy(
                    e2_hbm.at[idx2b.at[slot, sb]],
                    rows_v.at[slot, pl.ds(BW + sb * SB, SB)], sem))
            return cps

        def transpose_select(slot):
            for g in range(BW // L):
                rsel16 = rselb[slot, pl.ds(g * L, L)]
                sb, o = g // (SB // L), (g % (SB // L)) * L
                vv = rsel16.astype(jnp.float32)
                for c in range(D):
                    obuf_v[slot, c // 8, sb, c % 8, pl.ds(o, L)] = vv

        def fire_writes(h, slot):
            return [
                pltpu.async_copy(obuf_v.at[slot, cb],
                                 out_hbm.at[h, cb, pl.ds(wid * NSB, NSB)],
                                 wsem)
                for cb in range(D // 8)
            ]

        def body(p, carry):
            h0 = p * 2
            h1 = h0 + 1
            prep(h0, 0)
            g0 = fire_gathers(0, gsem0)
            prep(h1, 1)
            g1 = fire_gathers(1, gsem1)
            for cp in g0:
                cp.wait()
            transpose_select(0)
            w0 = fire_writes(h0, 0)
            for cp in g1:
                cp.wait()
            transpose_select(1)
            w1 = fire_writes(h1, 1)
            for w in w0 + w1:
                w.wait()
            return carry

        lax.fori_loop(0, HIST // 2, body, 0)

    return k


def kernel(inputs, e1, e2):
    bsz, hist = inputs.shape
    idx = inputs.reshape(bsz * hist).astype(jnp.int32)
    out5 = _embed_kernel()(idx, e1, e2)
    x = out5.transpose(2, 4, 0, 1, 3)
    return x.reshape(bsz, hist, D)


# speedup vs baseline: 2.5865x; 2.5806x over previous
import functools

import jax
import jax.numpy as jnp
from jax import lax
from jax.experimental import pallas as pl
from jax.experimental.pallas import tpu as pltpu
from jax.experimental.pallas import tpu_sc as plsc

INPUT_DIM = 1000000
HALF = INPUT_DIM // 2
D = 32

NC = 2
NS = 16
NW = NC * NS
L = 16

SEG = 12800
BLK = 128
CAP = SEG + 160


def _embed_kernel(B: int):
    n_per_w = B // NW
    n_seg = n_per_w // SEG

    mesh = plsc.VectorSubcoreMesh(core_axis_name="c", subcore_axis_name="s")

    @functools.partial(
        pl.kernel,
        out_type=jax.ShapeDtypeStruct((B, D), jnp.float32),
        mesh=mesh,
        compiler_params=pltpu.CompilerParams(use_tc_tiling_on_sc=False,
                                             needs_layout_passes=False),
        scratch_types=[
            pltpu.VMEM((SEG,), jnp.int32),
            pltpu.VMEM((CAP,), jnp.int32),
            pltpu.VMEM((CAP,), jnp.int32),
            pltpu.VMEM((CAP,), jnp.int32),
            pltpu.VMEM((CAP,), jnp.int32),
            pltpu.VMEM((2, BLK, D), jnp.float32),
            pltpu.VMEM((2, BLK), jnp.int32),
            pltpu.SemaphoreType.DMA,
            pltpu.SemaphoreType.DMA,
        ],
    )
    def k(idx_hbm, e1_hbm, e2_hbm, out_hbm,
          idx_v, idxb1, posb1, idxb2, posb2, rows_v, pstage, gsem, ssem):
        wid = lax.axis_index("s") * NC + lax.axis_index("c")
        wbase = wid * n_per_w
        iota = lax.broadcasted_iota(jnp.int32, (L,), 0)

        def run_table(idxb, posb, cnt, table_hbm):
            @pl.when(cnt > 0)
            def _():
                last = cnt - 1
                li = plsc.load_gather(idxb, [jnp.full((L,), last, jnp.int32)])
                lp = plsc.load_gather(posb, [jnp.full((L,), last, jnp.int32)])
                g0 = cnt - (cnt & (L - 1))
                keep = (g0 + iota) < cnt
                idxb[pl.ds(g0, L)] = jnp.where(keep, idxb[pl.ds(g0, L)], li)
                posb[pl.ds(g0, L)] = jnp.where(keep, posb[pl.ds(g0, L)], lp)
                for t in range(1, BLK // L + 1):
                    idxb[pl.ds(g0 + t * L, L)] = li
                    posb[pl.ds(g0 + t * L, L)] = lp

                nb = (cnt + BLK - 1) // BLK

                def fire_gather(b, slot):
                    return pltpu.async_copy(
                        table_hbm.at[idxb.at[pl.ds(b * BLK, BLK)]],
                        rows_v.at[slot], gsem)

                def stage_and_scatter(b, slot):
                    for t in range(BLK // L):
                        pstage[slot, pl.ds(t * L, L)] = (
                            posb[pl.ds(b * BLK + t * L, L)])
                    return pltpu.async_copy(
                        rows_v.at[slot], out_hbm.at[pstage.at[slot]], ssem)

                def body(p, carry):
                    b = p * 2
                    ga = fire_gather(b, 0)
                    ga.wait()
                    sa = stage_and_scatter(b, 0)

                    @pl.when(b + 1 < nb)
                    def _():
                        gb = fire_gather(b + 1, 1)
                        gb.wait()
                        sb = stage_and_scatter(b + 1, 1)
                        sb.wait()

                    sa.wait()
                    return carry

                lax.fori_loop(0, (nb + 1) // 2, body, 0)

        for seg in range(n_seg):
            seg_gbase = wbase + seg * SEG
            pltpu.sync_copy(idx_hbm.at[pl.ds(seg_gbase, SEG)], idx_v)

            def compact(g, carry):
                c1, c2 = carry
                idx16 = idx_v[pl.ds(g * L, L)]
                m = idx16 < HALF
                n1 = plsc.all_reduce_population_count(m)[0]
                pos16 = (seg_gbase + g * L) + iota
                plsc.store_compressed(idxb1.at[pl.ds(c1, L)], idx16, mask=m)
                plsc.store_compressed(posb1.at[pl.ds(c1, L)], pos16, mask=m)
                plsc.store_compressed(idxb2.at[pl.ds(c2, L)], idx16 - HALF,
                                      mask=~m)
                plsc.store_compressed(posb2.at[pl.ds(c2, L)], pos16, mask=~m)
                return (c1 + n1, c2 + (L - n1))

            zero = jnp.int32(0)
            c1, c2 = lax.fori_loop(0, SEG // L, compact, (zero, zero))

            run_table(idxb1, posb1, c1, e1_hbm)
            run_table(idxb2, posb2, c2, e2_hbm)

    return k


def kernel(inputs, e1, e2):
    bsz, hist = inputs.shape
    B = bsz * hist
    idx = inputs.reshape(B).astype(jnp.int32)
    out = _embed_kernel(B)(idx, e1, e2)
    return out.reshape(bsz, hist, D)
